# Initial kernel scaffold; baseline (speedup 1.0000x reference)
import jax
import jax.numpy as jnp
from jax.experimental import pallas as pl


def kernel(user_idx, item_idx, qtus, rating_matrix, weight, b_user, b_item):
    raise NotImplementedError("write your pallas kernel here")


# speedup vs baseline: 1.8176x; 1.8176x over previous
import functools

import jax
import jax.numpy as jnp
from jax import lax
from jax.experimental import pallas as pl
from jax.experimental.pallas import tpu as pltpu
from jax.experimental.pallas import tpu_sc as plsc

L = 16


@functools.lru_cache(maxsize=None)
def _build(U, I, K, B):
    NC, NS = 2, 16
    NW = NC * NS
    assert B % (NW * L) == 0
    PB = B // NW
    PCH = PB // L
    NE = PB * K
    assert NE % L == 0
    ECH = NE // L
    IDX_MINOR = 128
    assert NE % IDX_MINOR == 0
    IDX_MAJOR = NE // IDX_MINOR

    mesh = plsc.VectorSubcoreMesh(core_axis_name="c", subcore_axis_name="s")

    @functools.partial(
        pl.kernel,
        out_type=jax.ShapeDtypeStruct((B,), jnp.float32),
        mesh=mesh,
        compiler_params=pltpu.CompilerParams(
            needs_layout_passes=False, use_tc_tiling_on_sc=False),
        scratch_types=[
            pltpu.VMEM((PB,), jnp.int32),
            pltpu.VMEM((PB,), jnp.int32),
            pltpu.VMEM((PB,), jnp.int32),
            pltpu.VMEM((PB,), jnp.float32),
            pltpu.VMEM((PB,), jnp.float32),
            pltpu.VMEM((U,), jnp.float32),
            pltpu.VMEM((I,), jnp.float32),
            pltpu.VMEM((IDX_MAJOR, IDX_MINOR), jnp.int32),
            pltpu.VMEM((IDX_MAJOR, IDX_MINOR), jnp.int32),
            pltpu.VMEM((IDX_MAJOR, IDX_MINOR), jnp.int32),
            pltpu.VMEM((IDX_MAJOR, IDX_MINOR), jnp.float32),
            pltpu.VMEM((IDX_MAJOR, IDX_MINOR), jnp.float32),
            pltpu.VMEM((IDX_MAJOR, IDX_MINOR), jnp.float32),
            pltpu.VMEM((K, PB), jnp.float32),
            pltpu.VMEM((PB,), jnp.float32),
            pltpu.SemaphoreType.DMA,
            pltpu.SemaphoreType.DMA,
        ],
    )
    def launch(uidx_hbm, tidx_hbm, qflat_hbm, rflat_hbm, wflat_hbm,
               buser_hbm, bitem_hbm, out_hbm,
               u_v, t_v, base_v, bu_v, bi_v, buser_v, bitem_v, q_v,
               widx_v, ridx_v, w_v, r_v, adj_v, t_prod, out_v, sem0, sem1):
        cid = lax.axis_index("c")
        sid = lax.axis_index("s")
        wid = sid * NC + cid
        base = wid * PB

        pltpu.sync_copy(uidx_hbm.at[pl.ds(base, PB)], u_v)
        pltpu.sync_copy(tidx_hbm.at[pl.ds(base, PB)], t_v)
        pltpu.sync_copy(buser_hbm, buser_v)
        pltpu.sync_copy(bitem_hbm, bitem_v)

        lanes = lax.iota(jnp.int32, L)

        def p1(i, _):
            sl = pl.ds(i * L, L)
            uvec = u_v[sl]
            tvec = t_v[sl]
            base_v[sl] = uvec * I + tvec
            bu_v[sl] = plsc.load_gather(buser_v, [uvec])
            bi_v[sl] = plsc.load_gather(bitem_v, [tvec])
            return 0

        lax.fori_loop(0, PCH, p1, 0)

        def p2(c, _):
            n_v = c * L + lanes
            jv = n_v // K
            kv = n_v - jv * K
            base_b = plsc.load_gather(base_v, [jv])
            p = c * L
            widx_v[p // IDX_MINOR, pl.ds(p % IDX_MINOR, L)] = base_b * K + kv
            return 0

        lax.fori_loop(0, ECH, p2, 0)
        qcopies = [
            pltpu.async_copy(qflat_hbm.at[widx_v.at[m]], q_v.at[m], sem0)
            for m in range(IDX_MAJOR)
        ]
        for cp in qcopies:
            cp.wait()

        def p3(c, _):
            p = c * L
            maj = p // IDX_MINOR
            sl = pl.ds(p % IDX_MINOR, L)
            n_v = c * L + lanes
            jv = n_v // K
            qv = q_v[maj, sl]
            t_b = plsc.load_gather(t_v, [jv])
            u_b = plsc.load_gather(u_v, [jv])
            bu_b = plsc.load_gather(bu_v, [jv])
            bj = plsc.load_gather(bitem_v, [qv])
            widx_v[maj, sl] = qv * I + t_b
            ridx_v[maj, sl] = u_b * I + qv
            adj_v[maj, sl] = bu_b + bj
            return 0

        lax.fori_loop(0, ECH, p3, 0)

        copies = []
        for m in range(IDX_MAJOR):
            copies.append(pltpu.async_copy(
                wflat_hbm.at[widx_v.at[m]], w_v.at[m], sem0))
            copies.append(pltpu.async_copy(
                rflat_hbm.at[ridx_v.at[m]], r_v.at[m], sem1))
        for cp in copies:
            cp.wait()

        def p5(c, _):
            p = c * L
            maj = p // IDX_MINOR
            sl = pl.ds(p % IDX_MINOR, L)
            prod = w_v[maj, sl] * (r_v[maj, sl] - adj_v[maj, sl])
            n_v = c * L + lanes
            jv = n_v // K
            kv = n_v - jv * K
            plsc.store_scatter(t_prod, [kv, jv], prod)
            return 0

        lax.fori_loop(0, ECH, p5, 0)

        def p6(o, _):
            sl = pl.ds(o * L, L)
            acc = t_prod[0, sl]
            for e in range(1, K):
                acc = acc + t_prod[e, sl]
            out_v[sl] = bu_v[sl] + bi_v[sl] + acc * (1.0 / K)
            return 0

        lax.fori_loop(0, PCH, p6, 0)
        pltpu.sync_copy(out_v, out_hbm.at[pl.ds(base, PB)])

    return launch


def kernel(user_idx, item_idx, qtus, rating_matrix, weight, b_user, b_item):
    U, I = rating_matrix.shape
    K = qtus.shape[-1]
    B = user_idx.shape[0]
    launch = _build(U, I, K, B)
    return launch(
        user_idx.astype(jnp.int32),
        item_idx.astype(jnp.int32),
        qtus.reshape(U * I * K),
        rating_matrix.reshape(U * I),
        weight.reshape(I * I),
        b_user,
        b_item,
    )


# speedup vs baseline: 5.7486x; 3.1627x over previous
import functools

import jax
import jax.numpy as jnp
from jax import lax
from jax.experimental import pallas as pl
from jax.experimental.pallas import tpu as pltpu
from jax.experimental.pallas import tpu_sc as plsc

L = 16


@functools.lru_cache(maxsize=None)
def _build(U, I, K, B):
    NC, NS = 2, 16
    NW = NC * NS
    assert B % (NW * L) == 0
    PB = B // NW
    PCH = PB // L
    NE = PB * K
    assert NE % L == 0
    ECH = NE // L
    IDX_MINOR = 128
    assert NE % IDX_MINOR == 0
    IDX_MAJOR = NE // IDX_MINOR

    mesh = plsc.VectorSubcoreMesh(core_axis_name="c", subcore_axis_name="s")

    @functools.partial(
        pl.kernel,
        out_type=jax.ShapeDtypeStruct((B,), jnp.float32),
        mesh=mesh,
        compiler_params=pltpu.CompilerParams(needs_layout_passes=False),
        scratch_types=[
            pltpu.VMEM((PB,), jnp.int32),
            pltpu.VMEM((PB,), jnp.int32),
            pltpu.VMEM((PB,), jnp.int32),
            pltpu.VMEM((PB,), jnp.float32),
            pltpu.VMEM((PB,), jnp.float32),
            pltpu.VMEM((U,), jnp.float32),
            pltpu.VMEM((I,), jnp.float32),
            pltpu.VMEM((PB, K), jnp.int32),
            pltpu.VMEM((IDX_MAJOR, IDX_MINOR), jnp.int32),
            pltpu.VMEM((IDX_MAJOR, IDX_MINOR), jnp.int32),
            pltpu.VMEM((IDX_MAJOR, IDX_MINOR), jnp.float32),
            pltpu.VMEM((IDX_MAJOR, IDX_MINOR), jnp.float32),
            pltpu.VMEM((IDX_MAJOR, IDX_MINOR), jnp.float32),
            pltpu.VMEM((K, PB), jnp.float32),
            pltpu.VMEM((PB,), jnp.float32),
            pltpu.SemaphoreType.DMA,
            pltpu.SemaphoreType.DMA,
        ],
    )
    def launch(uidx_hbm, tidx_hbm, q2d_hbm, rflat_hbm, wflat_hbm,
               buser_hbm, bitem_hbm, out_hbm,
               u_v, t_v, base_v, bu_v, bi_v, buser_v, bitem_v, q_v,
               widx_v, ridx_v, w_v, r_v, adj_v, t_prod, out_v, sem0, sem1):
        cid = lax.axis_index("c")
        sid = lax.axis_index("s")
        wid = sid * NC + cid
        base = wid * PB

        pltpu.sync_copy(uidx_hbm.at[pl.ds(base, PB)], u_v)
        pltpu.sync_copy(tidx_hbm.at[pl.ds(base, PB)], t_v)
        pltpu.sync_copy(buser_hbm, buser_v)
        pltpu.sync_copy(bitem_hbm, bitem_v)

        lanes = lax.iota(jnp.int32, L)

        def p1(i, _):
            sl = pl.ds(i * L, L)
            uvec = u_v[sl]
            tvec = t_v[sl]
            base_v[sl] = uvec * I + tvec
            bu_v[sl] = plsc.load_gather(buser_v, [uvec])
            bi_v[sl] = plsc.load_gather(bitem_v, [tvec])
            return 0

        lax.fori_loop(0, PCH, p1, 0)

        qcopies = []
        for c in range(PCH):
            rvec = base_v[pl.ds(c * L, L)]
            for lane in range(L):
                j = c * L + lane
                qcopies.append(
                    pltpu.async_copy(q2d_hbm.at[rvec[lane]], q_v.at[j], sem0))
        for cp in qcopies:
            cp.wait()

        def p3(c, _):
            p = c * L
            maj = p // IDX_MINOR
            sl = pl.ds(p % IDX_MINOR, L)
            n_v = p + lanes
            jv = n_v // K
            kv = n_v - jv * K
            qv = plsc.load_gather(q_v, [jv, kv])
            t_b = plsc.load_gather(t_v, [jv])
            u_b = plsc.load_gather(u_v, [jv])
            bu_b = plsc.load_gather(bu_v, [jv])
            bj = plsc.load_gather(bitem_v, [qv])
            widx_v[maj, sl] = qv * I + t_b
            ridx_v[maj, sl] = u_b * I + qv
            adj_v[maj, sl] = bu_b + bj
            return 0

        lax.fori_loop(0, ECH, p3, 0)

        copies = []
        for m in range(IDX_MAJOR):
            copies.append(pltpu.async_copy(
                wflat_hbm.at[widx_v.at[m]], w_v.at[m], sem0))
            copies.append(pltpu.async_copy(
                rflat_hbm.at[ridx_v.at[m]], r_v.at[m], sem1))
        for cp in copies:
            cp.wait()

        def p5(c, _):
            p = c * L
            maj = p // IDX_MINOR
            sl = pl.ds(p % IDX_MINOR, L)
            prod = w_v[maj, sl] * (r_v[maj, sl] - adj_v[maj, sl])
            n_v = p + lanes
            jv = n_v // K
            kv = n_v - jv * K
            plsc.store_scatter(t_prod, [kv, jv], prod)
            return 0

        lax.fori_loop(0, ECH, p5, 0)

        def p6(o, _):
            sl = pl.ds(o * L, L)
            acc = t_prod[0, sl]
            for e in range(1, K):
                acc = acc + t_prod[e, sl]
            out_v[sl] = bu_v[sl] + bi_v[sl] + acc * (1.0 / K)
            return 0

        lax.fori_loop(0, PCH, p6, 0)
        pltpu.sync_copy(out_v, out_hbm.at[pl.ds(base, PB)])

    return launch


def kernel(user_idx, item_idx, qtus, rating_matrix, weight, b_user, b_item):
    U, I = rating_matrix.shape
    K = qtus.shape[-1]
    B = user_idx.shape[0]
    launch = _build(U, I, K, B)
    return launch(
        user_idx.astype(jnp.int32),
        item_idx.astype(jnp.int32),
        qtus.reshape(U * I, K),
        rating_matrix.reshape(U * I),
        weight.reshape(I * I),
        b_user,
        b_item,
    )
